# baseline (speedup 1.0000x reference)
import functools

import jax
import jax.numpy as jnp
from jax import lax
from jax.experimental import pallas as pl
from jax.experimental.pallas import tpu as pltpu
from jax.experimental.pallas import tpu_sc as plsc

N_NODES = 100000
D = 128
N_SEG = 256
NC = 2
NS = 16
W = NC * NS
L = 16
CHUNK = 3128
LB_COLS = 272
SEARCH_ITERS = 12
SEG_PER_W = N_SEG // W
BLK = 256
BADV = BLK - 8

_mesh = plsc.VectorSubcoreMesh(
    core_axis_name="c", subcore_axis_name="s", num_cores=NC, num_subcores=NS
)
_params = pltpu.CompilerParams(needs_layout_passes=False)


def _wid():
    return lax.axis_index("s") * NC + lax.axis_index("c")


@functools.partial(
    pl.kernel,
    out_type=jax.ShapeDtypeStruct((W * LB_COLS,), jnp.int32),
    mesh=_mesh,
    compiler_params=_params,
    scratch_types=[
        pltpu.VMEM((CHUNK,), jnp.int32),
        pltpu.VMEM((LB_COLS,), jnp.int32),
    ],
)
def _stage1(seg_hbm, lb_hbm, ids_v, row_v):
    w = _wid()
    tlo = w * CHUNK
    thi = jnp.minimum(tlo + CHUNK, N_NODES)
    cs = pl.multiple_of(jnp.minimum(tlo, N_NODES - CHUNK), 8)
    pltpu.sync_copy(seg_hbm.at[pl.ds(cs, CHUNK)], ids_v)
    base = tlo - cs
    cnt = thi - tlo
    iota = lax.iota(jnp.int32, L)
    for v in range(LB_COLS // L):
        s = v * L + iota
        lo0 = jnp.full((L,), base, jnp.int32)
        size0 = jnp.full((L,), cnt, jnp.int32)

        def step(_, carry, s=s):
            lo, size = carry
            active = size > 0
            half = size // 2
            mid = lo + half
            val = plsc.load_gather(ids_v, [jnp.minimum(mid, CHUNK - 1)])
            pred = active & (val < s)
            lo = jnp.where(pred, mid + 1, lo)
            size = jnp.where(active, jnp.where(pred, size - half - 1, half), size)
            return lo, size

        lo, _unused = lax.fori_loop(0, SEARCH_ITERS, step, (lo0, size0))
        row_v[pl.ds(v * L, L)] = lo - base
    pltpu.sync_copy(row_v, lb_hbm.at[pl.ds(pl.multiple_of(w * LB_COLS, 8), LB_COLS)])


@functools.partial(
    pl.kernel,
    out_type=jax.ShapeDtypeStruct((N_SEG, D), jnp.float32),
    mesh=_mesh,
    compiler_params=_params,
    scratch_types=[
        pltpu.VMEM((W * LB_COLS,), jnp.int32),
        pltpu.VMEM((BLK, D), jnp.float32),
        pltpu.VMEM((BLK, D), jnp.float32),
        pltpu.VMEM((SEG_PER_W, D), jnp.float32),
        pltpu.VMEM((SEG_PER_W, D), jnp.float32),
        pltpu.SemaphoreType.DMA,
        pltpu.SemaphoreType.DMA,
    ],
)
def _stage2(feat_hbm, lb_hbm, out_hbm, lb_v, fbuf0, fbuf1, acc_v, out_v, sem0, sem1):
    w = _wid()
    pltpu.sync_copy(lb_hbm, lb_v)
    iota = lax.iota(jnp.int32, L)
    col = SEG_PER_W * w + iota

    def sum_row(r, acc):
        return acc + plsc.load_gather(lb_v, [r * LB_COLS + col])

    bsum = lax.fori_loop(0, W, sum_row, jnp.zeros((L,), jnp.int32))
    b = [jnp.sum(jnp.where(iota == j, bsum, 0)) for j in range(SEG_PER_W + 1)]

    zf = jnp.zeros((L,), jnp.float32)
    for j in range(SEG_PER_W):
        for c in range(D // L):
            acc_v[j, pl.ds(c * L, L)] = zf

    b_lo, b_hi = b[0], b[SEG_PER_W]
    nblk = (b_hi - b_lo + BADV - 1) // BADV

    def blk_start(k):
        blk_lo = b_lo + k * BADV
        return pl.multiple_of(jnp.minimum((blk_lo // 8) * 8, N_NODES - BLK), 8)

    def dma_desc(k, fb, sem):
        return pltpu.make_async_copy(
            feat_hbm.at[pl.ds(blk_start(k), BLK)], fb, sem
        )

    def process(k, fb):
        blk_lo = b_lo + k * BADV
        blk_hi = jnp.minimum(blk_lo + BADV, b_hi)
        start = blk_start(k)
        for j in range(SEG_PER_W):
            lo = jnp.maximum(b[j], blk_lo)
            hi = jnp.minimum(b[j + 1], blk_hi)

            @pl.when(hi > lo)
            def _(j=j, lo=lo, hi=hi, start=start, fb=fb):
                ilo = lo - start
                ihi = hi - start

                def row_body(i, accs):
                    return tuple(
                        accs[c] + fb[i, pl.ds(c * L, L)] for c in range(D // L)
                    )

                accs = plsc.parallel_loop(
                    ilo, ihi, 1, unroll=4, carry=tuple(zf for _ in range(D // L))
                )(row_body)
                for c in range(D // L):
                    plsc.addupdate(acc_v.at[j, pl.ds(c * L, L)], accs[c])

    @pl.when(nblk > 0)
    def _():
        dma_desc(0, fbuf0, sem0).start()

    def pair_body(p, carry):
        k0 = 2 * p

        @pl.when(k0 < nblk)
        def _():
            dma_desc(k0, fbuf0, sem0).wait()

            @pl.when(k0 + 1 < nblk)
            def _():
                dma_desc(k0 + 1, fbuf1, sem1).start()

            process(k0, fbuf0)

        @pl.when(k0 + 1 < nblk)
        def _():
            dma_desc(k0 + 1, fbuf1, sem1).wait()

            @pl.when(k0 + 2 < nblk)
            def _():
                dma_desc(k0 + 2, fbuf0, sem0).start()

            process(k0 + 1, fbuf1)

        return carry

    lax.fori_loop(0, (nblk + 1) // 2, pair_body, 0)

    for j in range(SEG_PER_W):
        cntv = jnp.full((L,), b[j + 1] - b[j], jnp.int32).astype(jnp.float32)
        rec = 1.0 / jnp.maximum(cntv, 1.0)
        for c in range(D // L):
            out_v[j, pl.ds(c * L, L)] = acc_v[j, pl.ds(c * L, L)] * rec
    pltpu.sync_copy(
        out_v, out_hbm.at[pl.ds(pl.multiple_of(SEG_PER_W * w, 8), SEG_PER_W)]
    )


def kernel(feat, segment_ids):
    seg = segment_ids.astype(jnp.int32)
    lb = _stage1(seg)
    return _stage2(feat, lb)


# speedup vs baseline: 1.1785x; 1.1785x over previous
import functools

import jax
import jax.numpy as jnp
from jax import lax
from jax.experimental import pallas as pl
from jax.experimental.pallas import tpu as pltpu
from jax.experimental.pallas import tpu_sc as plsc

N_NODES = 100000
D = 128
N_SEG = 256
NC = 2
NS = 16
W = NC * NS
L = 16
TPC = N_NODES // NS
CHUNK = 6264
LB_COLS = 272
SEARCH_ITERS = 13
SEG_PER_W = N_SEG // W
BLK = 256
BADV = BLK - 8

_mesh = plsc.VectorSubcoreMesh(
    core_axis_name="c", subcore_axis_name="s", num_cores=NC, num_subcores=NS
)
_params = pltpu.CompilerParams(needs_layout_passes=False)


@functools.partial(
    pl.kernel,
    out_type=jax.ShapeDtypeStruct((N_SEG, D), jnp.float32),
    mesh=_mesh,
    compiler_params=_params,
    scratch_types=[
        pltpu.VMEM((CHUNK,), jnp.int32),
        pltpu.VMEM((LB_COLS,), jnp.int32),
        pltpu.VMEM((NS * LB_COLS,), jnp.int32),
        pltpu.VMEM((BLK, D), jnp.float32),
        pltpu.VMEM((BLK, D), jnp.float32),
        pltpu.VMEM((SEG_PER_W, D), jnp.float32),
        pltpu.VMEM((SEG_PER_W, D), jnp.float32),
        pltpu.VMEM_SHARED((NS * LB_COLS,), jnp.int32),
        pltpu.SemaphoreType.DMA,
        pltpu.SemaphoreType.DMA,
    ],
)
def _fused(
    seg_hbm, feat_hbm, out_hbm,
    ids_v, row_v, lb_v, fbuf0, fbuf1, acc_v, out_v, shared_lb, sem0, sem1,
):
    cidx = lax.axis_index("c")
    sidx = lax.axis_index("s")
    w = sidx * NC + cidx
    iota = lax.iota(jnp.int32, L)

    tlo = sidx * TPC
    cs = pl.multiple_of(jnp.minimum((tlo // 8) * 8, N_NODES - CHUNK), 8)
    pltpu.sync_copy(seg_hbm.at[pl.ds(cs, CHUNK)], ids_v)
    base = tlo - cs
    for v in range(LB_COLS // L):
        sv = v * L + iota
        lo0 = jnp.full((L,), base, jnp.int32)
        size0 = jnp.full((L,), TPC, jnp.int32)

        def step(_, carry, sv=sv):
            lo, size = carry
            active = size > 0
            half = size // 2
            mid = lo + half
            val = plsc.load_gather(ids_v, [jnp.minimum(mid, CHUNK - 1)])
            pred = active & (val < sv)
            lo = jnp.where(pred, mid + 1, lo)
            size = jnp.where(active, jnp.where(pred, size - half - 1, half), size)
            return lo, size

        lo, _unused = lax.fori_loop(0, SEARCH_ITERS, step, (lo0, size0))
        row_v[pl.ds(v * L, L)] = lo - base
    pltpu.sync_copy(
        row_v, shared_lb.at[pl.ds(pl.multiple_of(sidx * LB_COLS, 8), LB_COLS)]
    )
    plsc.subcore_barrier()
    pltpu.sync_copy(shared_lb, lb_v)

    col = SEG_PER_W * w + iota

    def sum_row(r, acc):
        return acc + plsc.load_gather(lb_v, [r * LB_COLS + col])

    bsum = lax.fori_loop(0, NS, sum_row, jnp.zeros((L,), jnp.int32))
    b = [jnp.sum(jnp.where(iota == j, bsum, 0)) for j in range(SEG_PER_W + 1)]

    zf = jnp.zeros((L,), jnp.float32)
    for j in range(SEG_PER_W):
        for c in range(D // L):
            acc_v[j, pl.ds(c * L, L)] = zf

    b_lo, b_hi = b[0], b[SEG_PER_W]
    nblk = (b_hi - b_lo + BADV - 1) // BADV

    def blk_start(k):
        blk_lo = b_lo + k * BADV
        return pl.multiple_of(jnp.minimum((blk_lo // 8) * 8, N_NODES - BLK), 8)

    def dma_desc(k, fb, sem):
        return pltpu.make_async_copy(
            feat_hbm.at[pl.ds(blk_start(k), BLK)], fb, sem
        )

    def process(k, fb):
        blk_lo = b_lo + k * BADV
        blk_hi = jnp.minimum(blk_lo + BADV, b_hi)
        start = blk_start(k)
        for j in range(SEG_PER_W):
            lo = jnp.maximum(b[j], blk_lo)
            hi = jnp.minimum(b[j + 1], blk_hi)

            @pl.when(hi > lo)
            def _(j=j, lo=lo, hi=hi, start=start, fb=fb):
                ilo = lo - start
                ihi = hi - start

                def row_body(i, accs):
                    return tuple(
                        accs[c] + fb[i, pl.ds(c * L, L)] for c in range(D // L)
                    )

                accs = lax.fori_loop(
                    ilo, ihi, row_body, tuple(zf for _ in range(D // L))
                )
                for c in range(D // L):
                    plsc.addupdate(acc_v.at[j, pl.ds(c * L, L)], accs[c])

    @pl.when(nblk > 0)
    def _():
        dma_desc(0, fbuf0, sem0).start()

    def pair_body(p, carry):
        k0 = 2 * p

        @pl.when(k0 < nblk)
        def _():
            dma_desc(k0, fbuf0, sem0).wait()

            @pl.when(k0 + 1 < nblk)
            def _():
                dma_desc(k0 + 1, fbuf1, sem1).start()

            process(k0, fbuf0)

        @pl.when(k0 + 1 < nblk)
        def _():
            dma_desc(k0 + 1, fbuf1, sem1).wait()

            @pl.when(k0 + 2 < nblk)
            def _():
                dma_desc(k0 + 2, fbuf0, sem0).start()

            process(k0 + 1, fbuf1)

        return carry

    lax.fori_loop(0, (nblk + 1) // 2, pair_body, 0)

    for j in range(SEG_PER_W):
        cntv = jnp.full((L,), b[j + 1] - b[j], jnp.int32).astype(jnp.float32)
        rec = 1.0 / jnp.maximum(cntv, 1.0)
        for c in range(D // L):
            out_v[j, pl.ds(c * L, L)] = acc_v[j, pl.ds(c * L, L)] * rec
    pltpu.sync_copy(
        out_v, out_hbm.at[pl.ds(pl.multiple_of(SEG_PER_W * w, 8), SEG_PER_W)]
    )


def kernel(feat, segment_ids):
    seg = segment_ids.astype(jnp.int32)
    return _fused(seg, feat)


# speedup vs baseline: 1.2347x; 1.0477x over previous
import functools

import jax
import jax.numpy as jnp
from jax import lax
from jax.experimental import pallas as pl
from jax.experimental.pallas import tpu as pltpu
from jax.experimental.pallas import tpu_sc as plsc

N_NODES = 100000
D = 128
N_SEG = 256
NC = 2
NS = 16
W = NC * NS
L = 16
TPC = N_NODES // NS
CHUNK = 6264
LB_COLS = 272
SEARCH_ITERS = 13
SEG_PER_W = N_SEG // W
BLK = 384
BADV = BLK - 8

_mesh = plsc.VectorSubcoreMesh(
    core_axis_name="c", subcore_axis_name="s", num_cores=NC, num_subcores=NS
)
_params = pltpu.CompilerParams(needs_layout_passes=False)


@functools.partial(
    pl.kernel,
    out_type=jax.ShapeDtypeStruct((N_SEG, D), jnp.float32),
    mesh=_mesh,
    compiler_params=_params,
    scratch_types=[
        pltpu.VMEM((CHUNK,), jnp.int32),
        pltpu.VMEM((LB_COLS,), jnp.int32),
        pltpu.VMEM((NS * LB_COLS,), jnp.int32),
        pltpu.VMEM((BLK, D), jnp.float32),
        pltpu.VMEM((BLK, D), jnp.float32),
        pltpu.VMEM((SEG_PER_W, D), jnp.float32),
        pltpu.VMEM((SEG_PER_W, D), jnp.float32),
        pltpu.VMEM_SHARED((NS * LB_COLS,), jnp.int32),
        pltpu.SemaphoreType.DMA,
        pltpu.SemaphoreType.DMA,
    ],
)
def _fused(
    seg_hbm, feat_hbm, out_hbm,
    ids_v, row_v, lb_v, fbuf0, fbuf1, acc_v, out_v, shared_lb, sem0, sem1,
):
    cidx = lax.axis_index("c")
    sidx = lax.axis_index("s")
    w = sidx * NC + cidx
    iota = lax.iota(jnp.int32, L)

    tlo = sidx * TPC
    cs = pl.multiple_of(jnp.minimum((tlo // 8) * 8, N_NODES - CHUNK), 8)
    pltpu.sync_copy(seg_hbm.at[pl.ds(cs, CHUNK)], ids_v)
    base = tlo - cs
    for v in range(LB_COLS // L):
        sv = v * L + iota
        lo0 = jnp.full((L,), base, jnp.int32)
        size0 = jnp.full((L,), TPC, jnp.int32)

        def step(_, carry, sv=sv):
            lo, size = carry
            active = size > 0
            half = size // 2
            mid = lo + half
            val = plsc.load_gather(ids_v, [jnp.minimum(mid, CHUNK - 1)])
            pred = active & (val < sv)
            lo = jnp.where(pred, mid + 1, lo)
            size = jnp.where(active, jnp.where(pred, size - half - 1, half), size)
            return lo, size

        lo, _unused = lax.fori_loop(0, SEARCH_ITERS, step, (lo0, size0))
        row_v[pl.ds(v * L, L)] = lo - base
    pltpu.sync_copy(
        row_v, shared_lb.at[pl.ds(pl.multiple_of(sidx * LB_COLS, 8), LB_COLS)]
    )
    plsc.subcore_barrier()
    pltpu.sync_copy(shared_lb, lb_v)

    col = SEG_PER_W * w + iota

    def sum_row(r, acc):
        return acc + plsc.load_gather(lb_v, [r * LB_COLS + col])

    bsum = lax.fori_loop(0, NS, sum_row, jnp.zeros((L,), jnp.int32))
    b = [jnp.sum(jnp.where(iota == j, bsum, 0)) for j in range(SEG_PER_W + 1)]

    zf = jnp.zeros((L,), jnp.float32)
    for j in range(SEG_PER_W):
        for c in range(D // L):
            acc_v[j, pl.ds(c * L, L)] = zf

    b_lo, b_hi = b[0], b[SEG_PER_W]
    nblk = (b_hi - b_lo + BADV - 1) // BADV

    def blk_start(k):
        blk_lo = b_lo + k * BADV
        return pl.multiple_of(jnp.minimum((blk_lo // 8) * 8, N_NODES - BLK), 8)

    def dma_desc(k, fb, sem):
        return pltpu.make_async_copy(
            feat_hbm.at[pl.ds(blk_start(k), BLK)], fb, sem
        )

    def process(k, fb):
        blk_lo = b_lo + k * BADV
        blk_hi = jnp.minimum(blk_lo + BADV, b_hi)
        start = blk_start(k)
        for j in range(SEG_PER_W):
            lo = jnp.maximum(b[j], blk_lo)
            hi = jnp.minimum(b[j + 1], blk_hi)

            @pl.when(hi > lo)
            def _(j=j, lo=lo, hi=hi, start=start, fb=fb):
                ilo = lo - start
                ihi = hi - start

                def row_body(i, accs):
                    return tuple(
                        accs[c] + fb[i, pl.ds(c * L, L)] for c in range(D // L)
                    )

                accs = lax.fori_loop(
                    ilo, ihi, row_body, tuple(zf for _ in range(D // L))
                )
                for c in range(D // L):
                    plsc.addupdate(acc_v.at[j, pl.ds(c * L, L)], accs[c])

    @pl.when(nblk > 0)
    def _():
        dma_desc(0, fbuf0, sem0).start()

    def pair_body(p, carry):
        k0 = 2 * p

        @pl.when(k0 < nblk)
        def _():
            dma_desc(k0, fbuf0, sem0).wait()

            @pl.when(k0 + 1 < nblk)
            def _():
                dma_desc(k0 + 1, fbuf1, sem1).start()

            process(k0, fbuf0)

        @pl.when(k0 + 1 < nblk)
        def _():
            dma_desc(k0 + 1, fbuf1, sem1).wait()

            @pl.when(k0 + 2 < nblk)
            def _():
                dma_desc(k0 + 2, fbuf0, sem0).start()

            process(k0 + 1, fbuf1)

        return carry

    lax.fori_loop(0, (nblk + 1) // 2, pair_body, 0)

    for j in range(SEG_PER_W):
        cntv = jnp.full((L,), b[j + 1] - b[j], jnp.int32).astype(jnp.float32)
        rec = 1.0 / jnp.maximum(cntv, 1.0)
        for c in range(D // L):
            out_v[j, pl.ds(c * L, L)] = acc_v[j, pl.ds(c * L, L)] * rec
    pltpu.sync_copy(
        out_v, out_hbm.at[pl.ds(pl.multiple_of(SEG_PER_W * w, 8), SEG_PER_W)]
    )


def kernel(feat, segment_ids):
    seg = segment_ids.astype(jnp.int32)
    return _fused(seg, feat)
